# Initial kernel scaffold; baseline (speedup 1.0000x reference)
import jax
import jax.numpy as jnp
from jax.experimental import pallas as pl


def kernel(x, inc_idx, conv0_W, conv0_b, conv1_W, conv1_b, mlp0_W, mlp0_b, mlp1_W, mlp1_b):
    raise NotImplementedError("write your pallas kernel here")


# speedup vs baseline: 8.5251x; 8.5251x over previous
import functools

import jax
import jax.numpy as jnp
from jax import lax
from jax.experimental import pallas as pl
from jax.experimental.pallas import tpu as pltpu
from jax.experimental.pallas import tpu_sc as plsc

N = 10000
E = 10000
NNZ = 320000
F = 128

NC, NS = 2, 16
NW = NC * NS
NNZ_W = NNZ // NW
N_PAD = 10240
RPT = N_PAD // NS

MBLK = 1000
GRID = N // MBLK

_SC_MESH = dict(core_axis_name="c", subcore_axis_name="s",
                num_cores=NC, num_subcores=NS)


CH = 80
NCHW = NNZ_W // CH


def _sc_counts(nidx, eidx, ones_rows, zf):
    mesh = plsc.VectorSubcoreMesh(**_SC_MESH)

    @functools.partial(
        pl.kernel, mesh=mesh,
        out_type=(jax.ShapeDtypeStruct((NC * N_PAD, F), jnp.float32),
                  jax.ShapeDtypeStruct((NC * N_PAD, F), jnp.float32)),
        scratch_types=[
            pltpu.VMEM((CH,), jnp.int32),
            pltpu.VMEM((CH, F), jnp.float32),
            pltpu.VMEM_SHARED((N_PAD, F), jnp.float32),
        ],
    )
    def k(nidx_hbm, eidx_hbm, ones_hbm, zf_hbm, outd_hbm, outb_hbm,
          idx_v, ones_v, acc_sh):
        c = lax.axis_index("c")
        s = lax.axis_index("s")
        wid = s * NC + c
        base = wid * NNZ_W
        pltpu.sync_copy(ones_hbm, ones_v)

        def phase(idx_hbm, out_hbm):
            pltpu.sync_copy(zf_hbm, acc_sh.at[pl.ds(s * RPT, RPT)])
            plsc.subcore_barrier()

            def body(j, carry):
                pltpu.sync_copy(idx_hbm.at[pl.ds(base + j * CH, CH)], idx_v)
                pltpu.sync_copy(ones_v, acc_sh.at[idx_v], add=True)
                return carry

            lax.fori_loop(0, NCHW, body, 0)
            plsc.subcore_barrier()
            pltpu.sync_copy(acc_sh.at[pl.ds(s * RPT, RPT)],
                            out_hbm.at[pl.ds(c * N_PAD + s * RPT, RPT)])

        phase(nidx_hbm, outd_hbm)
        plsc.subcore_barrier()
        phase(eidx_hbm, outb_hbm)

    return k(nidx, eidx, ones_rows, zf)


def _sc_segsum(src, gidx, sidx, zf):
    mesh = plsc.VectorSubcoreMesh(**_SC_MESH)

    @functools.partial(
        pl.kernel, mesh=mesh,
        out_type=jax.ShapeDtypeStruct((NC * N_PAD, F), jnp.float32),
        scratch_types=[
            pltpu.VMEM((CH,), jnp.int32),
            pltpu.VMEM((CH,), jnp.int32),
            pltpu.VMEM((CH, F), jnp.float32),
            pltpu.VMEM_SHARED((N_PAD, F), jnp.float32),
            pltpu.SemaphoreType.DMA,
        ],
    )
    def k(src_hbm, gidx_hbm, sidx_hbm, zf_hbm, out_hbm,
          gi_v, si_v, rows_v, acc_sh, sem):
        c = lax.axis_index("c")
        s = lax.axis_index("s")
        wid = s * NC + c
        base = wid * NNZ_W
        pltpu.sync_copy(zf_hbm, acc_sh.at[pl.ds(s * RPT, RPT)])
        plsc.subcore_barrier()

        def body(j, carry):
            pltpu.sync_copy(gidx_hbm.at[pl.ds(base + j * CH, CH)], gi_v)
            pltpu.sync_copy(sidx_hbm.at[pl.ds(base + j * CH, CH)], si_v)
            pltpu.async_copy(src_hbm.at[gi_v], rows_v, sem).wait()
            pltpu.sync_copy(rows_v, acc_sh.at[si_v], add=True)
            return carry

        lax.fori_loop(0, NCHW, body, 0)
        plsc.subcore_barrier()
        pltpu.sync_copy(acc_sh.at[pl.ds(s * RPT, RPT)],
                        out_hbm.at[pl.ds(c * N_PAD + s * RPT, RPT)])

    return k(src, gidx, sidx, zf)


def _inv_from_counts(c_ref):
    cnt = c_ref[0][:, 0:1] + c_ref[1][:, 0:1]
    return jnp.where(cnt > 0, 1.0 / jnp.where(cnt > 0, cnt, 1.0), 0.0)


def _tc_mm(x, w):
    def body(x_ref, w_ref, o_ref):
        o_ref[...] = jnp.dot(x_ref[...], w_ref[...],
                             preferred_element_type=jnp.float32)

    return pl.pallas_call(
        body,
        grid=(GRID,),
        in_specs=[pl.BlockSpec((MBLK, F), lambda i: (i, 0)),
                  pl.BlockSpec((F, F), lambda i: (0, 0))],
        out_specs=pl.BlockSpec((MBLK, F), lambda i: (i, 0)),
        out_shape=jax.ShapeDtypeStruct((N, F), jnp.float32),
    )(x, w)


def _tc_combine_scale(p, cnt):
    def body(p_ref, c_ref, o_ref):
        o_ref[...] = (p_ref[0] + p_ref[1]) * _inv_from_counts(c_ref)

    return pl.pallas_call(
        body,
        grid=(GRID,),
        in_specs=[pl.BlockSpec((NC, MBLK, F), lambda i: (0, i, 0)),
                  pl.BlockSpec((NC, MBLK, F), lambda i: (0, i, 0))],
        out_specs=pl.BlockSpec((MBLK, F), lambda i: (i, 0)),
        out_shape=jax.ShapeDtypeStruct((N, F), jnp.float32),
    )(p.reshape(NC, N_PAD, F), cnt.reshape(NC, N_PAD, F))


def _tc_combine_scale_bias_mm(q, cnt, b, w):
    def body(q_ref, c_ref, b_ref, w_ref, o_ref):
        t = (q_ref[0] + q_ref[1]) * _inv_from_counts(c_ref) + b_ref[...]
        o_ref[...] = jnp.dot(t, w_ref[...], preferred_element_type=jnp.float32)

    return pl.pallas_call(
        body,
        grid=(GRID,),
        in_specs=[pl.BlockSpec((NC, MBLK, F), lambda i: (0, i, 0)),
                  pl.BlockSpec((NC, MBLK, F), lambda i: (0, i, 0)),
                  pl.BlockSpec((1, F), lambda i: (0, 0)),
                  pl.BlockSpec((F, F), lambda i: (0, 0))],
        out_specs=pl.BlockSpec((MBLK, F), lambda i: (i, 0)),
        out_shape=jax.ShapeDtypeStruct((N, F), jnp.float32),
    )(q.reshape(NC, N_PAD, F), cnt.reshape(NC, N_PAD, F), b.reshape(1, F), w)


def _tc_final(q, cnt, b, w0, b0, w1, b1):
    def body(q_ref, c_ref, b_ref, w0_ref, b0_ref, w1_ref, b1_ref, o_ref):
        z = (q_ref[0] + q_ref[1]) * _inv_from_counts(c_ref) + b_ref[...]
        h = jnp.maximum(
            jnp.dot(z, w0_ref[...], preferred_element_type=jnp.float32)
            + b0_ref[...], 0.0)
        logits = jnp.dot(h, w1_ref[...],
                         preferred_element_type=jnp.float32) + b1_ref[...]
        m = jnp.max(logits, axis=1, keepdims=True)
        ex = jnp.exp(logits - m)
        o_ref[...] = ex / jnp.sum(ex, axis=1, keepdims=True)

    return pl.pallas_call(
        body,
        grid=(GRID,),
        in_specs=[pl.BlockSpec((NC, MBLK, F), lambda i: (0, i, 0)),
                  pl.BlockSpec((NC, MBLK, F), lambda i: (0, i, 0)),
                  pl.BlockSpec((1, F), lambda i: (0, 0)),
                  pl.BlockSpec((F, 64), lambda i: (0, 0)),
                  pl.BlockSpec((1, 64), lambda i: (0, 0)),
                  pl.BlockSpec((64, 8), lambda i: (0, 0)),
                  pl.BlockSpec((1, 8), lambda i: (0, 0))],
        out_specs=pl.BlockSpec((MBLK, 8), lambda i: (i, 0)),
        out_shape=jax.ShapeDtypeStruct((N, 8), jnp.float32),
    )(q.reshape(NC, N_PAD, F), cnt.reshape(NC, N_PAD, F), b.reshape(1, F),
      w0, b0.reshape(1, 64), w1, b1.reshape(1, 8))


def kernel(x, inc_idx, conv0_W, conv0_b, conv1_W, conv1_b,
           mlp0_W, mlp0_b, mlp1_W, mlp1_b):
    nidx = inc_idx[0]
    eidx = inc_idx[1]
    ones_rows = jnp.ones((CH, F), jnp.float32)
    zf = jnp.zeros((RPT, F), jnp.float32)

    cnt_d, cnt_b = _sc_counts(nidx, eidx, ones_rows, zf)

    h = _tc_mm(x, conv0_W)
    p = _sc_segsum(h, nidx, eidx, zf)
    ef = _tc_combine_scale(p, cnt_b)
    q = _sc_segsum(ef, eidx, nidx, zf)
    h = _tc_combine_scale_bias_mm(q, cnt_d, conv0_b, conv1_W)

    p = _sc_segsum(h, nidx, eidx, zf)
    ef = _tc_combine_scale(p, cnt_b)
    q = _sc_segsum(ef, eidx, nidx, zf)
    return _tc_final(q, cnt_d, conv1_b, mlp0_W, mlp0_b, mlp1_W, mlp1_b)
